# baseline (speedup 1.0000x reference)
import functools

import jax
import jax.numpy as jnp
from jax import lax
from jax.experimental import pallas as pl
from jax.experimental.pallas import tpu as pltpu
from jax.experimental.pallas import tpu_sc as plsc

N_VOCAB = 1000000
D_MODEL = 64
BATCH = 4096
SEQ = 200

_NC = 2
_NS = 16
_NW = _NC * _NS

_B = BATCH * SEQ
_CH = 128
_BPW = _B // _NW
_NCHUNK = _BPW // _CH

_SLAB = 128
_KROWS = 3908
_K = _KROWS * _SLAB
_CFULL = 3903
_PITCH = 2 * D_MODEL + 1


@functools.partial(
    pl.kernel,
    out_type=jax.ShapeDtypeStruct((_K, 2 * D_MODEL), jnp.float32),
    mesh=plsc.VectorSubcoreMesh(core_axis_name="c", subcore_axis_name="s"),
    compiler_params=pltpu.CompilerParams(needs_layout_passes=False),
    scratch_types=[
        pltpu.VMEM((D_MODEL, _SLAB), jnp.float32),
        pltpu.VMEM((D_MODEL, _SLAB), jnp.float32),
        pltpu.VMEM((_SLAB, _PITCH), jnp.float32),
        pltpu.SemaphoreType.DMA,
    ],
)
def _sc_transpose(w_hbm, tab_hbm, in0_v, in1_v, buf_v, sem):
    wid = lax.axis_index("s") * _NC + lax.axis_index("c")
    nk = 122 + jnp.where(wid < _KROWS - 122 * _NW, 1, 0)
    iota = lax.iota(jnp.int32, 16)

    def chunk(k, _):
        c = wid + k * _NW
        s1 = jnp.minimum(c * _SLAB + _K, (N_VOCAB // _SLAB) * _SLAB)
        s1 = pl.multiple_of(s1, _SLAB)
        pltpu.async_copy(w_hbm.at[:, pl.ds(c * _SLAB, _SLAB)], in0_v,
                         sem).wait()
        pltpu.async_copy(w_hbm.at[:, pl.ds(s1, _SLAB)], in1_v, sem).wait()

        def row(d, dv):
            for g in range(_SLAB // 16):
                rows = g * 16 + iota
                plsc.store_scatter(buf_v, [rows, dv],
                                   in0_v[d, pl.ds(g * 16, 16)])
                plsc.store_scatter(buf_v, [rows, dv + D_MODEL],
                                   in1_v[d, pl.ds(g * 16, 16)])
            return dv + 1

        lax.fori_loop(0, D_MODEL, row, jnp.zeros((16,), jnp.int32))
        pltpu.sync_copy(buf_v.at[:, pl.ds(0, 2 * D_MODEL)],
                        tab_hbm.at[pl.ds(c * _SLAB, _SLAB)])
        return _

    lax.fori_loop(0, nk, chunk, None)


@functools.partial(
    pl.kernel,
    out_type=jax.ShapeDtypeStruct((_B, D_MODEL), jnp.float32),
    mesh=plsc.VectorSubcoreMesh(core_axis_name="c", subcore_axis_name="s"),
    compiler_params=pltpu.CompilerParams(use_tc_tiling_on_sc=False),
    scratch_types=[
        pltpu.VMEM((_NCHUNK, _CH), jnp.int32),
        pltpu.VMEM((_CH,), jnp.int32),
        pltpu.VMEM((_CH, D_MODEL), jnp.float32),
        pltpu.SemaphoreType.DMA,
    ],
)
def _sc_gather(table_hbm, idx_hbm, out_hbm, idx_v, ridx_v, out_v, sem):
    wid = lax.axis_index("s") * _NC + lax.axis_index("c")
    base = wid * _BPW
    pltpu.sync_copy(idx_hbm.at[wid], idx_v)

    def chunk(c, _):
        for g in range(_CH // 16):
            xv = idx_v[c, pl.ds(g * 16, 16)]
            hi = xv >= _K
            ridx_v[pl.ds(g * 16, 16)] = jnp.where(
                hi, 2 * (xv - _K) + 1, 2 * xv)
        pltpu.async_copy(table_hbm.at[ridx_v], out_v, sem).wait()
        pltpu.sync_copy(out_v, out_hbm.at[pl.ds(base + c * _CH, _CH)])
        return _

    lax.fori_loop(0, _NCHUNK, chunk, None)


def kernel(x, W_embed):
    table = _sc_transpose(W_embed).reshape(2 * _K, D_MODEL)
    idx = x.astype(jnp.int32).reshape(_NW, _NCHUNK, _CH)
    out = _sc_gather(table, idx)
    return out.reshape(BATCH, SEQ, D_MODEL)


# speedup vs baseline: 1.1777x; 1.1777x over previous
import functools

import jax
import jax.numpy as jnp
from jax import lax
from jax.experimental import pallas as pl
from jax.experimental.pallas import tpu as pltpu
from jax.experimental.pallas import tpu_sc as plsc

N_VOCAB = 1000000
D_MODEL = 64
BATCH = 4096
SEQ = 200

_NC = 2
_NS = 16
_NW = _NC * _NS

_B = BATCH * SEQ
_CH = 128
_BPW = _B // _NW
_NCHUNK = _BPW // _CH

_SLAB = 128
_KROWS = 3908
_K = _KROWS * _SLAB
_CFULL = 3903
_PITCH = 2 * D_MODEL + 1


@functools.partial(
    pl.kernel,
    out_type=jax.ShapeDtypeStruct((_K, 2 * D_MODEL), jnp.float32),
    mesh=plsc.VectorSubcoreMesh(core_axis_name="c", subcore_axis_name="s"),
    compiler_params=pltpu.CompilerParams(needs_layout_passes=False),
    scratch_types=[
        pltpu.VMEM((D_MODEL, _SLAB), jnp.float32),
        pltpu.VMEM((D_MODEL, _SLAB), jnp.float32),
        pltpu.VMEM((_SLAB, _PITCH), jnp.float32),
        pltpu.SemaphoreType.DMA,
    ],
)
def _sc_transpose(w_hbm, tab_hbm, in0_v, in1_v, buf_v, sem):
    wid = lax.axis_index("s") * _NC + lax.axis_index("c")
    nk = 122 + jnp.where(wid < _KROWS - 122 * _NW, 1, 0)
    iota = lax.iota(jnp.int32, 16)

    def chunk(k, _):
        c = wid + k * _NW
        s1 = jnp.minimum(c * _SLAB + _K, (N_VOCAB // _SLAB) * _SLAB)
        s1 = pl.multiple_of(s1, _SLAB)
        pltpu.async_copy(w_hbm.at[:, pl.ds(c * _SLAB, _SLAB)], in0_v,
                         sem).wait()
        pltpu.async_copy(w_hbm.at[:, pl.ds(s1, _SLAB)], in1_v, sem).wait()

        @plsc.parallel_loop(0, D_MODEL, carry=jnp.zeros((16,), jnp.int32),
                            unroll=4)
        def row(d, dv):
            for g in range(_SLAB // 16):
                rows = g * 16 + iota
                plsc.store_scatter(buf_v, [rows, dv],
                                   in0_v[d, pl.ds(g * 16, 16)])
                plsc.store_scatter(buf_v, [rows, dv + D_MODEL],
                                   in1_v[d, pl.ds(g * 16, 16)])
            return dv + 1
        pltpu.sync_copy(buf_v.at[:, pl.ds(0, 2 * D_MODEL)],
                        tab_hbm.at[pl.ds(c * _SLAB, _SLAB)])
        return _

    lax.fori_loop(0, nk, chunk, None)


@functools.partial(
    pl.kernel,
    out_type=jax.ShapeDtypeStruct((_B, D_MODEL), jnp.float32),
    mesh=plsc.VectorSubcoreMesh(core_axis_name="c", subcore_axis_name="s"),
    compiler_params=pltpu.CompilerParams(use_tc_tiling_on_sc=False),
    scratch_types=[
        pltpu.VMEM((_NCHUNK, _CH), jnp.int32),
        pltpu.VMEM((_CH,), jnp.int32),
        pltpu.VMEM((_CH, D_MODEL), jnp.float32),
        pltpu.SemaphoreType.DMA,
    ],
)
def _sc_gather(table_hbm, idx_hbm, out_hbm, idx_v, ridx_v, out_v, sem):
    wid = lax.axis_index("s") * _NC + lax.axis_index("c")
    base = wid * _BPW
    pltpu.sync_copy(idx_hbm.at[wid], idx_v)

    def chunk(c, _):
        for g in range(_CH // 16):
            xv = idx_v[c, pl.ds(g * 16, 16)]
            hi = xv >= _K
            ridx_v[pl.ds(g * 16, 16)] = jnp.where(
                hi, 2 * (xv - _K) + 1, 2 * xv)
        pltpu.async_copy(table_hbm.at[ridx_v], out_v, sem).wait()
        pltpu.sync_copy(out_v, out_hbm.at[pl.ds(base + c * _CH, _CH)])
        return _

    lax.fori_loop(0, _NCHUNK, chunk, None)


def kernel(x, W_embed):
    table = _sc_transpose(W_embed).reshape(2 * _K, D_MODEL)
    idx = x.astype(jnp.int32).reshape(_NW, _NCHUNK, _CH)
    out = _sc_gather(table, idx)
    return out.reshape(BATCH, SEQ, D_MODEL)


# speedup vs baseline: 1.6896x; 1.4347x over previous
import functools

import jax
import jax.numpy as jnp
from jax import lax
from jax.experimental import pallas as pl
from jax.experimental.pallas import tpu as pltpu
from jax.experimental.pallas import tpu_sc as plsc

N_VOCAB = 1000000
D_MODEL = 64
BATCH = 4096
SEQ = 200

_NC = 2
_NS = 16
_NW = _NC * _NS

_B = BATCH * SEQ
_CH = 128
_BPW = _B // _NW
_NCHUNK = _BPW // _CH


@functools.partial(
    pl.kernel,
    out_type=jax.ShapeDtypeStruct((_B, D_MODEL), jnp.float32),
    mesh=plsc.VectorSubcoreMesh(core_axis_name="c", subcore_axis_name="s"),
    compiler_params=pltpu.CompilerParams(use_tc_tiling_on_sc=False),
    scratch_types=[
        pltpu.VMEM((_NCHUNK, _CH), jnp.int32),
        pltpu.VMEM((_CH, D_MODEL), jnp.float32),
        pltpu.SemaphoreType.DMA,
    ],
)
def _sc_gather(table_hbm, idx_hbm, out_hbm, idx_v, out_v, sem):
    wid = lax.axis_index("s") * _NC + lax.axis_index("c")
    base = wid * _BPW
    pltpu.sync_copy(idx_hbm.at[wid], idx_v)

    def chunk(c, _):
        pltpu.async_copy(table_hbm.at[idx_v.at[c]], out_v, sem).wait()
        pltpu.sync_copy(out_v, out_hbm.at[pl.ds(base + c * _CH, _CH)])
        return _

    lax.fori_loop(0, _NCHUNK, chunk, None)


def kernel(x, W_embed):
    table = jnp.swapaxes(W_embed, 0, 1)
    idx = x.astype(jnp.int32).reshape(_NW, _NCHUNK, _CH)
    out = _sc_gather(table, idx)
    return out.reshape(BATCH, SEQ, D_MODEL)


# speedup vs baseline: 2.3294x; 1.3786x over previous
import functools

import jax
import jax.numpy as jnp
from jax import lax
from jax.experimental import pallas as pl
from jax.experimental.pallas import tpu as pltpu
from jax.experimental.pallas import tpu_sc as plsc

N_VOCAB = 1000000
D_MODEL = 64
BATCH = 4096
SEQ = 200

_NC = 2
_NS = 16
_NW = _NC * _NS

_B = BATCH * SEQ
_CH = 128
_BPW = _B // _NW
_NCHUNK = _BPW // _CH

_VB = 2048
_NVB = 245
_K = _VB * _NVB
_MAXB = (N_VOCAB - 1) // _VB


def _transpose_body(a_ref, b_ref, out_ref):
    out_ref[:, 0:D_MODEL] = a_ref[...].T
    out_ref[:, D_MODEL:2 * D_MODEL] = b_ref[...].T


def _build_table(w):
    return pl.pallas_call(
        _transpose_body,
        grid=(_NVB,),
        in_specs=[
            pl.BlockSpec((D_MODEL, _VB), lambda i: (0, i)),
            pl.BlockSpec((D_MODEL, _VB),
                         lambda i: (0, jnp.minimum(i + _NVB, _MAXB))),
        ],
        out_specs=pl.BlockSpec((_VB, 2 * D_MODEL), lambda i: (i, 0)),
        out_shape=jax.ShapeDtypeStruct((_K, 2 * D_MODEL), jnp.float32),
    )(w, w)


@functools.partial(
    pl.kernel,
    out_type=jax.ShapeDtypeStruct((_B, D_MODEL), jnp.float32),
    mesh=plsc.VectorSubcoreMesh(core_axis_name="c", subcore_axis_name="s"),
    compiler_params=pltpu.CompilerParams(use_tc_tiling_on_sc=False),
    scratch_types=[
        pltpu.VMEM((_NCHUNK, _CH), jnp.int32),
        pltpu.VMEM((2, _CH), jnp.int32),
        pltpu.VMEM((2, _CH, D_MODEL), jnp.float32),
        pltpu.SemaphoreType.DMA,
    ],
)
def _sc_gather(table_hbm, idx_hbm, out_hbm, idx_v, ridx_v, out_v, sem):
    wid = lax.axis_index("s") * _NC + lax.axis_index("c")
    base = wid * _BPW
    pltpu.sync_copy(idx_hbm.at[wid], idx_v)

    def start_gather(c, b):
        for g in range(_CH // 16):
            xv = idx_v[c, pl.ds(g * 16, 16)]
            hi = xv >= _K
            ridx_v[b, pl.ds(g * 16, 16)] = jnp.where(
                hi, 2 * (xv - _K) + 1, 2 * xv)
        pltpu.async_copy(table_hbm.at[ridx_v.at[b]], out_v.at[b], sem)

    start_gather(0, 0)

    def chunk(c, _):
        b = c % 2
        @pl.when(c + 1 < _NCHUNK)
        def _():
            start_gather(c + 1, 1 - b)
        pltpu.make_async_copy(out_hbm.at[pl.ds(base, _CH)], out_v.at[b],
                              sem).wait()
        pltpu.sync_copy(out_v.at[b],
                        out_hbm.at[pl.ds(base + c * _CH, _CH)])
        return _

    lax.fori_loop(0, _NCHUNK, chunk, None)


def kernel(x, W_embed):
    table = _build_table(W_embed).reshape(2 * _K, D_MODEL)
    idx = x.astype(jnp.int32).reshape(_NW, _NCHUNK, _CH)
    out = _sc_gather(table, idx)
    return out.reshape(BATCH, SEQ, D_MODEL)


# speedup vs baseline: 2.4915x; 1.0696x over previous
import functools

import jax
import jax.numpy as jnp
from jax import lax
from jax.experimental import pallas as pl
from jax.experimental.pallas import tpu as pltpu
from jax.experimental.pallas import tpu_sc as plsc

N_VOCAB = 1000000
D_MODEL = 64
BATCH = 4096
SEQ = 200

_NC = 2
_NS = 16
_NW = _NC * _NS

_B = BATCH * SEQ
_CH = 128
_BPW = _B // _NW
_NCHUNK = _BPW // _CH

_VB = 4096
_NVB = 123
_K = _VB * _NVB
_MAXB = (N_VOCAB - 1) // _VB


def _transpose_body(a_ref, b_ref, out_ref):
    out_ref[:, 0:D_MODEL] = a_ref[...].T
    out_ref[:, D_MODEL:2 * D_MODEL] = b_ref[...].T


def _build_table(w):
    return pl.pallas_call(
        _transpose_body,
        grid=(_NVB,),
        in_specs=[
            pl.BlockSpec((D_MODEL, _VB), lambda i: (0, i)),
            pl.BlockSpec((D_MODEL, _VB),
                         lambda i: (0, jnp.minimum(i + _NVB, _MAXB))),
        ],
        out_specs=pl.BlockSpec((_VB, 2 * D_MODEL), lambda i: (i, 0)),
        out_shape=jax.ShapeDtypeStruct((_K, 2 * D_MODEL), jnp.float32),
    )(w, w)


@functools.partial(
    pl.kernel,
    out_type=jax.ShapeDtypeStruct((_B, D_MODEL), jnp.float32),
    mesh=plsc.VectorSubcoreMesh(core_axis_name="c", subcore_axis_name="s"),
    compiler_params=pltpu.CompilerParams(use_tc_tiling_on_sc=False),
    scratch_types=[
        pltpu.VMEM((_NCHUNK, _CH), jnp.int32),
        pltpu.VMEM((2, _CH), jnp.int32),
        pltpu.VMEM((2, _CH, D_MODEL), jnp.float32),
        pltpu.SemaphoreType.DMA,
    ],
)
def _sc_gather(table_hbm, idx_hbm, out_hbm, idx_v, ridx_v, out_v, sem):
    wid = lax.axis_index("s") * _NC + lax.axis_index("c")
    base = wid * _BPW
    pltpu.sync_copy(idx_hbm.at[wid], idx_v)

    def start_gather(c, b):
        for g in range(_CH // 16):
            xv = idx_v[c, pl.ds(g * 16, 16)]
            hi = xv >= _K
            ridx_v[b, pl.ds(g * 16, 16)] = jnp.where(
                hi, 2 * (xv - _K) + 1, 2 * xv)
        pltpu.async_copy(table_hbm.at[ridx_v.at[b]], out_v.at[b], sem)

    start_gather(0, 0)

    def chunk(c, _):
        b = c % 2
        @pl.when(c + 1 < _NCHUNK)
        def _():
            start_gather(c + 1, 1 - b)
        pltpu.make_async_copy(out_hbm.at[pl.ds(base, _CH)], out_v.at[b],
                              sem).wait()
        pltpu.sync_copy(out_v.at[b],
                        out_hbm.at[pl.ds(base + c * _CH, _CH)])
        return _

    lax.fori_loop(0, _NCHUNK, chunk, None)


def kernel(x, W_embed):
    table = _build_table(W_embed).reshape(2 * _K, D_MODEL)
    idx = x.astype(jnp.int32).reshape(_NW, _NCHUNK, _CH)
    out = _sc_gather(table, idx)
    return out.reshape(BATCH, SEQ, D_MODEL)


# speedup vs baseline: 2.5910x; 1.0400x over previous
import functools

import jax
import jax.numpy as jnp
from jax import lax
from jax.experimental import pallas as pl
from jax.experimental.pallas import tpu as pltpu
from jax.experimental.pallas import tpu_sc as plsc

N_VOCAB = 1000000
D_MODEL = 64
BATCH = 4096
SEQ = 200

_NC = 2
_NS = 16
_NW = _NC * _NS

_B = BATCH * SEQ
_CH = 128
_BPW = _B // _NW
_NCHUNK = _BPW // _CH

_VB = 8192
_NVB = 62
_K = _VB * _NVB
_MAXB = (N_VOCAB - 1) // _VB


def _transpose_body(a_ref, b_ref, out_ref):
    out_ref[:, 0:D_MODEL] = a_ref[...].T
    out_ref[:, D_MODEL:2 * D_MODEL] = b_ref[...].T


def _build_table(w):
    return pl.pallas_call(
        _transpose_body,
        grid=(_NVB,),
        in_specs=[
            pl.BlockSpec((D_MODEL, _VB), lambda i: (0, i)),
            pl.BlockSpec((D_MODEL, _VB),
                         lambda i: (0, jnp.minimum(i + _NVB, _MAXB))),
        ],
        out_specs=pl.BlockSpec((_VB, 2 * D_MODEL), lambda i: (i, 0)),
        out_shape=jax.ShapeDtypeStruct((_K, 2 * D_MODEL), jnp.float32),
    )(w, w)


@functools.partial(
    pl.kernel,
    out_type=jax.ShapeDtypeStruct((_B, D_MODEL), jnp.float32),
    mesh=plsc.VectorSubcoreMesh(core_axis_name="c", subcore_axis_name="s"),
    compiler_params=pltpu.CompilerParams(use_tc_tiling_on_sc=False),
    scratch_types=[
        pltpu.VMEM((_NCHUNK, _CH), jnp.int32),
        pltpu.VMEM((2, _CH), jnp.int32),
        pltpu.VMEM((2, _CH, D_MODEL), jnp.float32),
        pltpu.SemaphoreType.DMA,
    ],
)
def _sc_gather(table_hbm, idx_hbm, out_hbm, idx_v, ridx_v, out_v, sem):
    wid = lax.axis_index("s") * _NC + lax.axis_index("c")
    base = wid * _BPW
    pltpu.sync_copy(idx_hbm.at[wid], idx_v)

    def start_gather(c, b):
        for g in range(_CH // 16):
            xv = idx_v[c, pl.ds(g * 16, 16)]
            hi = xv >= _K
            ridx_v[b, pl.ds(g * 16, 16)] = jnp.where(
                hi, 2 * (xv - _K) + 1, 2 * xv)
        pltpu.async_copy(table_hbm.at[ridx_v.at[b]], out_v.at[b], sem)

    start_gather(0, 0)

    def chunk(c, _):
        b = c % 2
        @pl.when(c + 1 < _NCHUNK)
        def _():
            start_gather(c + 1, 1 - b)
        pltpu.make_async_copy(out_hbm.at[pl.ds(base, _CH)], out_v.at[b],
                              sem).wait()
        pltpu.sync_copy(out_v.at[b],
                        out_hbm.at[pl.ds(base + c * _CH, _CH)])
        return _

    lax.fori_loop(0, _NCHUNK, chunk, None)


def kernel(x, W_embed):
    table = _build_table(W_embed).reshape(2 * _K, D_MODEL)
    idx = x.astype(jnp.int32).reshape(_NW, _NCHUNK, _CH)
    out = _sc_gather(table, idx)
    return out.reshape(BATCH, SEQ, D_MODEL)


# speedup vs baseline: 2.6283x; 1.0144x over previous
import functools

import jax
import jax.numpy as jnp
from jax import lax
from jax.experimental import pallas as pl
from jax.experimental.pallas import tpu as pltpu
from jax.experimental.pallas import tpu_sc as plsc

N_VOCAB = 1000000
D_MODEL = 64
BATCH = 4096
SEQ = 200

_NC = 2
_NS = 16
_NW = _NC * _NS

_B = BATCH * SEQ
_CH = 128
_BPW = _B // _NW
_NCHUNK = _BPW // _CH

_VB = 16384
_NVB = 31
_K = _VB * _NVB
_MAXB = (N_VOCAB - 1) // _VB


def _transpose_body(a_ref, b_ref, out_ref):
    out_ref[:, 0:D_MODEL] = a_ref[...].T
    out_ref[:, D_MODEL:2 * D_MODEL] = b_ref[...].T


def _build_table(w):
    return pl.pallas_call(
        _transpose_body,
        grid=(_NVB,),
        in_specs=[
            pl.BlockSpec((D_MODEL, _VB), lambda i: (0, i)),
            pl.BlockSpec((D_MODEL, _VB),
                         lambda i: (0, jnp.minimum(i + _NVB, _MAXB))),
        ],
        out_specs=pl.BlockSpec((_VB, 2 * D_MODEL), lambda i: (i, 0)),
        out_shape=jax.ShapeDtypeStruct((_K, 2 * D_MODEL), jnp.float32),
    )(w, w)


@functools.partial(
    pl.kernel,
    out_type=jax.ShapeDtypeStruct((_B, D_MODEL), jnp.float32),
    mesh=plsc.VectorSubcoreMesh(core_axis_name="c", subcore_axis_name="s"),
    compiler_params=pltpu.CompilerParams(use_tc_tiling_on_sc=False),
    scratch_types=[
        pltpu.VMEM((_NCHUNK, _CH), jnp.int32),
        pltpu.VMEM((2, _CH), jnp.int32),
        pltpu.VMEM((2, _CH, D_MODEL), jnp.float32),
        pltpu.SemaphoreType.DMA,
    ],
)
def _sc_gather(table_hbm, idx_hbm, out_hbm, idx_v, ridx_v, out_v, sem):
    wid = lax.axis_index("s") * _NC + lax.axis_index("c")
    base = wid * _BPW
    pltpu.sync_copy(idx_hbm.at[wid], idx_v)

    def start_gather(c, b):
        for g in range(_CH // 16):
            xv = idx_v[c, pl.ds(g * 16, 16)]
            hi = xv >= _K
            ridx_v[b, pl.ds(g * 16, 16)] = jnp.where(
                hi, 2 * (xv - _K) + 1, 2 * xv)
        pltpu.async_copy(table_hbm.at[ridx_v.at[b]], out_v.at[b], sem)

    start_gather(0, 0)

    def chunk(c, _):
        b = c % 2
        @pl.when(c + 1 < _NCHUNK)
        def _():
            start_gather(c + 1, 1 - b)
        pltpu.make_async_copy(out_hbm.at[pl.ds(base, _CH)], out_v.at[b],
                              sem).wait()
        pltpu.sync_copy(out_v.at[b],
                        out_hbm.at[pl.ds(base + c * _CH, _CH)])
        return _

    lax.fori_loop(0, _NCHUNK, chunk, None)


def kernel(x, W_embed):
    table = _build_table(W_embed).reshape(2 * _K, D_MODEL)
    idx = x.astype(jnp.int32).reshape(_NW, _NCHUNK, _CH)
    out = _sc_gather(table, idx)
    return out.reshape(BATCH, SEQ, D_MODEL)
